# baseline (speedup 1.0000x reference)
import functools

import jax
import jax.numpy as jnp
from jax import lax
from jax.experimental import pallas as pl
from jax.experimental.pallas import tpu as pltpu
from jax.experimental.pallas import tpu_sc as plsc

B, L, H, SPAN = 32, 256, 256, 4
NC, NS = 2, 16


def _sc_body(yf_hbm, ybflat_hbm, hl_hbm, hg_hbm, zf_hbm, gb_hbm,
             rows_v, hl_v, hg_v, acc_v, sem):
    c = lax.axis_index("c")
    s = lax.axis_index("s")
    b = s * NC + c
    pltpu.sync_copy(hg_hbm.at[b], hg_v)
    pltpu.sync_copy(hl_hbm.at[b], hl_v)
    pltpu.sync_copy(yf_hbm.at[b], acc_v)
    lane = lax.iota(jnp.int32, 16)
    for k in range(2):
        pltpu.sync_copy(yf_hbm.at[b].at[pl.ds(k * 128, 128)], rows_v)
        for g in range(8):
            hvec = hl_v[pl.ds(k * 128 + g * 16, 16)]
            jvec = lane + (g * 16)

            def col_body(ci, _, hvec=hvec, jvec=jvec):
                cvec = jnp.full((16,), ci, dtype=jnp.int32)
                vals = plsc.load_gather(rows_v, [jvec, cvec])
                plsc.addupdate_scatter(acc_v, [hvec, cvec], vals)
                return 0

            lax.fori_loop(0, H, col_body, 0)
    pltpu.sync_copy(acc_v, zf_hbm.at[b])
    for k in range(2):
        pltpu.async_copy(ybflat_hbm.at[hg_v.at[k]], rows_v, sem).wait()
        pltpu.sync_copy(rows_v, gb_hbm.at[b].at[pl.ds(k * 128, 128)])


_sc_adj = functools.partial(
    pl.kernel,
    out_type=[jax.ShapeDtypeStruct((B, L, H), jnp.float32),
              jax.ShapeDtypeStruct((B, L, H), jnp.float32)],
    mesh=plsc.VectorSubcoreMesh(core_axis_name="c", subcore_axis_name="s"),
    compiler_params=pltpu.CompilerParams(
        use_tc_tiling_on_sc=False, needs_layout_passes=False),
    scratch_types=[
        pltpu.VMEM((128, H), jnp.float32),
        pltpu.VMEM((L,), jnp.int32),
        pltpu.VMEM((2, 128), jnp.int32),
        pltpu.VMEM((L, H), jnp.float32),
        pltpu.SemaphoreType.DMA,
    ],
)(_sc_body)


def _sc_stage(y_fw, y_bw, heads_local, heads_gather):
    return _sc_adj(y_fw, y_bw.reshape(B * L, H), heads_local, heads_gather)


def _tc1_body(x_ref, wf_ref, wb_ref, bf_ref, bb_ref, yf_ref, yb_ref):
    x = x_ref[0]
    yf_ref[0] = jnp.dot(x, wf_ref[...], preferred_element_type=jnp.float32) + bf_ref[...]
    yb_ref[0] = jnp.dot(x, wb_ref[...], preferred_element_type=jnp.float32) + bb_ref[...]


def _tc2_body(zf_ref, yb_ref, gb_ref, wf_ref, wb_ref, bf_ref, bb_ref,
              yf1_ref, yb1_ref):
    x = jnp.concatenate(
        [jnp.maximum(zf_ref[0], 0.0), jnp.maximum(yb_ref[0] + gb_ref[0], 0.0)],
        axis=1)
    yf1_ref[0] = jnp.dot(x, wf_ref[...], preferred_element_type=jnp.float32) + bf_ref[...]
    yb1_ref[0] = jnp.dot(x, wb_ref[...], preferred_element_type=jnp.float32) + bb_ref[...]


def _tc3_body(inst_ref, zf_ref, yb_ref, gb_ref, out_ref):
    b = pl.program_id(0)
    x = jnp.concatenate(
        [jnp.maximum(zf_ref[0], 0.0), jnp.maximum(yb_ref[0] + gb_ref[0], 0.0)],
        axis=1)
    s1 = inst_ref[4 * b + 0]
    s2 = inst_ref[4 * b + 2]
    col2 = lax.broadcasted_iota(jnp.int32, (2, L), 1)
    srow = jnp.where(lax.broadcasted_iota(jnp.int32, (2, L), 0) == 0, s1, s2)
    sel = jnp.where((col2 > srow) & (col2 <= srow + SPAN), 1.0 / SPAN, 0.0)
    res = jnp.dot(sel, x, preferred_element_type=jnp.float32)
    out_ref[0] = res.reshape(1, 4 * H)


def _batch_spec(shape_tail):
    return pl.BlockSpec((1,) + shape_tail, lambda b: (b,) + (0,) * len(shape_tail))


def _full_spec(shape):
    return pl.BlockSpec(shape, lambda b: (0,) * len(shape))


def _tc_matmul2(body, ins, w0, w1, b0, b1):
    act_specs = [_batch_spec((L, s.shape[-1])) for s in ins]
    return pl.pallas_call(
        body,
        grid=(B,),
        in_specs=act_specs + [
            _full_spec(w0.shape), _full_spec(w1.shape),
            _full_spec((1, H)), _full_spec((1, H)),
        ],
        out_specs=[_batch_spec((L, H)), _batch_spec((L, H))],
        out_shape=[jax.ShapeDtypeStruct((B, L, H), jnp.float32),
                   jax.ShapeDtypeStruct((B, L, H), jnp.float32)],
        compiler_params=pltpu.CompilerParams(dimension_semantics=("arbitrary",)),
    )(*ins, w0, w1, b0.reshape(1, H), b1.reshape(1, H))


@jax.jit
def kernel(word_h, heads, types, instances,
           W_fw0, W_bw0, W_fw1, W_bw1, b_fw0, b_bw0, b_fw1, b_bw1):
    del types
    heads_i = heads.astype(jnp.int32)
    heads_gather = (jnp.arange(B, dtype=jnp.int32)[:, None] * L
                    + heads_i).reshape(B, 2, 128)
    inst_flat = instances.astype(jnp.int32).reshape(B * 4)

    y_fw0, y_bw0 = _tc_matmul2(_tc1_body, [word_h], W_fw0, W_bw0, b_fw0, b_bw0)
    z_fw0, g_bw0 = _sc_stage(y_fw0, y_bw0, heads_i, heads_gather)
    y_fw1, y_bw1 = _tc_matmul2(_tc2_body, [z_fw0, y_bw0, g_bw0],
                               W_fw1, W_bw1, b_fw1, b_bw1)
    z_fw1, g_bw1 = _sc_stage(y_fw1, y_bw1, heads_i, heads_gather)

    grid_spec = pltpu.PrefetchScalarGridSpec(
        num_scalar_prefetch=1,
        grid=(B,),
        in_specs=[
            pl.BlockSpec((1, L, H), lambda b, inst: (b, 0, 0)),
            pl.BlockSpec((1, L, H), lambda b, inst: (b, 0, 0)),
            pl.BlockSpec((1, L, H), lambda b, inst: (b, 0, 0)),
        ],
        out_specs=pl.BlockSpec((1, 1, 4 * H), lambda b, inst: (b, 0, 0)),
    )
    out = pl.pallas_call(
        _tc3_body,
        grid_spec=grid_spec,
        out_shape=jax.ShapeDtypeStruct((B, 1, 4 * H), jnp.float32),
        compiler_params=pltpu.CompilerParams(dimension_semantics=("arbitrary",)),
    )(inst_flat, z_fw1, y_bw1, g_bw1)
    return out.reshape(B, 4 * H)
